# baseline (speedup 1.0000x reference)
import functools

import jax
import jax.numpy as jnp
from jax import lax
from jax.experimental import pallas as pl
from jax.experimental.pallas import tpu as pltpu
from jax.experimental.pallas import tpu_sc as plsc

N_BATCH = 4
N_CLASSES = 19
SP = 512 * 512
N_PIX = N_BATCH * SP
TOPK = int(0.8 * N_PIX)

N_SPLIT = 1
B_SPLIT = N_BATCH // N_SPLIT
PIX_SPLIT = N_PIX // N_SPLIT

NW = 32
CHUNK = PIX_SPLIT // NW
LANES = 16
VECS = CHUNK // LANES
BIN_SHIFT = 17
NBINS = 1 << (31 - BIN_SHIFT)
BIN_VECS = NBINS // LANES


R_CE = 256


def _ce_body(p_ref, t_ref, o_ref):
    p = p_ref[...]
    t = t_ref[...][:, None, :, :]
    m = jnp.max(p, axis=1, keepdims=True)
    s = jnp.sum(jnp.exp(p - m), axis=1, keepdims=True)
    cls = lax.broadcasted_iota(jnp.int32, p.shape, 1)
    pt = jnp.sum(jnp.where(cls == t, p, 0.0), axis=1, keepdims=True)
    o_ref[...] = (m + jnp.log(s) - pt)[:, 0, :, :]


_ce = pl.pallas_call(
    _ce_body,
    grid=(B_SPLIT, 512 // R_CE),
    in_specs=[
        pl.BlockSpec((1, N_CLASSES, R_CE, 512), lambda b, j: (b, 0, j, 0)),
        pl.BlockSpec((1, R_CE, 512), lambda b, j: (b, j, 0)),
    ],
    out_specs=pl.BlockSpec((1, R_CE, 512), lambda b, j: (b, j, 0)),
    out_shape=jax.ShapeDtypeStruct((B_SPLIT, 512, 512), jnp.float32),
    compiler_params=pltpu.CompilerParams(
        dimension_semantics=("parallel", "parallel")
    ),
)


N_SC = 2
N_SUB = 16
SLICE = NBINS // N_SUB


def _sc_hist_body(
    loss_hbm, cnt_hbm, sum_hbm, data_v, bins_v, ones_v, tmp_v, cnt_sh, sum_sh
):
    cid = lax.axis_index("c")
    sid = lax.axis_index("s")
    wid = sid * N_SC + cid
    pltpu.sync_copy(loss_hbm.at[pl.ds(wid * CHUNK, CHUNK)], data_v)

    ones = jnp.full((LANES,), 1.0, jnp.float32)
    izero = jnp.zeros((LANES,), jnp.int32)
    hunroll = 8

    def bins_body(i, carry):
        for u in range(hunroll):
            off = (i * hunroll + u) * LANES
            v = data_v[pl.ds(off, LANES)]
            bits = lax.bitcast_convert_type(v, jnp.int32)
            bins_v[pl.ds(off, LANES)] = lax.shift_right_logical(
                lax.max(bits, izero), BIN_SHIFT
            )
            ones_v[pl.ds(off, LANES)] = ones
        return carry

    lax.fori_loop(0, VECS // hunroll, bins_body, 0)

    zeros = jnp.zeros((LANES,), jnp.float32)

    def zero_body(i, carry):
        tmp_v[pl.ds(i * LANES, LANES)] = zeros
        return carry

    lax.fori_loop(0, SLICE // LANES, zero_body, 0)
    pltpu.sync_copy(tmp_v, cnt_sh.at[pl.ds(sid * SLICE, SLICE)])
    pltpu.sync_copy(tmp_v, sum_sh.at[pl.ds(sid * SLICE, SLICE)])
    plsc.subcore_barrier()

    pltpu.sync_copy(ones_v, cnt_sh.at[bins_v], add=True)
    pltpu.sync_copy(data_v, sum_sh.at[bins_v], add=True)
    plsc.subcore_barrier()

    pltpu.sync_copy(cnt_sh.at[pl.ds(sid * SLICE, SLICE)], tmp_v)
    pltpu.sync_copy(tmp_v, cnt_hbm.at[pl.ds(cid * NBINS + sid * SLICE, SLICE)])
    pltpu.sync_copy(sum_sh.at[pl.ds(sid * SLICE, SLICE)], tmp_v)
    pltpu.sync_copy(tmp_v, sum_hbm.at[pl.ds(cid * NBINS + sid * SLICE, SLICE)])


@functools.cache
def _sc_hist():
    return pl.kernel(
        _sc_hist_body,
        mesh=plsc.VectorSubcoreMesh(core_axis_name="c", subcore_axis_name="s"),
        out_type=[
            jax.ShapeDtypeStruct((N_SC * NBINS,), jnp.float32),
            jax.ShapeDtypeStruct((N_SC * NBINS,), jnp.float32),
        ],
        scratch_types=[
            pltpu.VMEM((CHUNK,), jnp.float32),
            pltpu.VMEM((CHUNK,), jnp.int32),
            pltpu.VMEM((CHUNK,), jnp.float32),
            pltpu.VMEM((SLICE,), jnp.float32),
            pltpu.VMEM_SHARED((NBINS,), jnp.float32),
            pltpu.VMEM_SHARED((NBINS,), jnp.float32),
        ],
        compiler_params=pltpu.CompilerParams(needs_layout_passes=False),
    )


def _cumsum_shift(x, axis):
    n = x.shape[axis]
    sh = 1
    while sh < n:
        zeros = lax.slice_in_dim(jnp.zeros_like(x), 0, sh, axis=axis)
        shifted = lax.slice_in_dim(x, 0, n - sh, axis=axis)
        x = x + lax.concatenate([zeros, shifted], dimension=axis)
        sh *= 2
    return x


def _sel_body(c0_ref, s0_ref, o_ref):
    c = jnp.sum(c0_ref[...], axis=0)
    s = jnp.sum(s0_ref[...], axis=0)
    ce0 = _cumsum_shift(c, 0) - c
    row_off = jnp.sum(ce0, axis=1, keepdims=True)
    pe = (_cumsum_shift(c, 1) - c) + row_off
    total = jnp.sum(c)
    suf = total - pe
    r = lax.broadcasted_iota(jnp.int32, c.shape, 0)
    l = lax.broadcasted_iota(jnp.int32, c.shape, 1)
    bidx = r * 128 + l
    kf = jnp.float32(TOPK)
    bsel = jnp.max(jnp.where(suf >= kf, bidx, -1))
    above = bidx > bsel
    at = bidx == bsel
    c_above = jnp.sum(jnp.where(above, c, 0.0))
    s_above = jnp.sum(jnp.where(above, s, 0.0))
    c_bin = jnp.sum(jnp.where(at, c, 0.0))
    s_bin = jnp.sum(jnp.where(at, s, 0.0))
    m = kf - c_above
    res = (s_above + m * (s_bin / jnp.maximum(c_bin, 1.0))) / kf
    o_ref[...] = jnp.broadcast_to(res, (1, 1))


_sel = pl.pallas_call(
    _sel_body,
    in_specs=[
        pl.BlockSpec((N_SC, NBINS // 128, 128), lambda: (0, 0, 0))
        for _ in range(2)
    ],
    out_specs=pl.BlockSpec((1, 1), lambda: (0, 0)),
    out_shape=jax.ShapeDtypeStruct((1, 1), jnp.float32),
)


def kernel(preds, targets):
    t = targets.astype(jnp.int32)
    loss = _ce(preds, t).reshape(N_PIX)
    cnt, sm = _sc_hist()(loss)
    out = _sel(
        cnt.reshape(N_SC, NBINS // 128, 128),
        sm.reshape(N_SC, NBINS // 128, 128),
    )
    return out[0, 0]


# speedup vs baseline: 1.2342x; 1.2342x over previous
import functools

import jax
import jax.numpy as jnp
from jax import lax
from jax.experimental import pallas as pl
from jax.experimental.pallas import tpu as pltpu
from jax.experimental.pallas import tpu_sc as plsc

N_BATCH = 4
N_CLASSES = 19
SP = 512 * 512
N_PIX = N_BATCH * SP
TOPK = int(0.8 * N_PIX)

N_SPLIT = 1
B_SPLIT = N_BATCH // N_SPLIT
PIX_SPLIT = N_PIX // N_SPLIT

NW = 32
CHUNK = PIX_SPLIT // NW
LANES = 16
VECS = CHUNK // LANES
BIN_SHIFT = 18
NBINS = 1 << (31 - BIN_SHIFT)
BIN_VECS = NBINS // LANES


R_CE = 256


def _ce_body(p_ref, t_ref, o_ref):
    p = p_ref[...]
    t = t_ref[...][:, None, :, :]
    m = jnp.max(p, axis=1, keepdims=True)
    s = jnp.sum(jnp.exp(p - m), axis=1, keepdims=True)
    cls = lax.broadcasted_iota(jnp.int32, p.shape, 1)
    pt = jnp.sum(jnp.where(cls == t, p, 0.0), axis=1, keepdims=True)
    o_ref[...] = (m + jnp.log(s) - pt)[:, 0, :, :]


_ce = pl.pallas_call(
    _ce_body,
    grid=(B_SPLIT, 512 // R_CE),
    in_specs=[
        pl.BlockSpec((1, N_CLASSES, R_CE, 512), lambda b, j: (b, 0, j, 0)),
        pl.BlockSpec((1, R_CE, 512), lambda b, j: (b, j, 0)),
    ],
    out_specs=pl.BlockSpec((1, R_CE, 512), lambda b, j: (b, j, 0)),
    out_shape=jax.ShapeDtypeStruct((B_SPLIT, 512, 512), jnp.float32),
    compiler_params=pltpu.CompilerParams(
        dimension_semantics=("parallel", "parallel")
    ),
)


def _sc_hist_body(
    loss_hbm, cnt_hbm, sum_hbm, data_v, cnt_v, sum_v, cnt2_v, sum2_v
):
    wid = lax.axis_index("s") * 2 + lax.axis_index("c")
    pltpu.sync_copy(loss_hbm.at[pl.ds(wid * CHUNK, CHUNK)], data_v)

    zeros = jnp.zeros((LANES,), jnp.float32)
    zunroll = 16

    def zero_body(i, carry):
        for u in range(zunroll):
            off = (i * zunroll + u) * LANES
            cnt_v[pl.ds(off, LANES)] = zeros
            sum_v[pl.ds(off, LANES)] = zeros
            cnt2_v[pl.ds(off, LANES)] = zeros
            sum2_v[pl.ds(off, LANES)] = zeros
        return carry

    lax.fori_loop(0, BIN_VECS // zunroll, zero_body, 0)

    ones = jnp.full((LANES,), 1.0, jnp.float32)
    izero = jnp.zeros((LANES,), jnp.int32)
    hunroll = 8

    def hist_body(i, carry):
        for u in range(hunroll):
            off = (i * hunroll + u) * LANES
            v = data_v[pl.ds(off, LANES)]
            bits = lax.bitcast_convert_type(v, jnp.int32)
            bins = lax.shift_right_logical(lax.max(bits, izero), BIN_SHIFT)
            if u % 2 == 0:
                plsc.addupdate_scatter(cnt_v, [bins], ones)
                plsc.addupdate_scatter(sum_v, [bins], v)
            else:
                plsc.addupdate_scatter(cnt2_v, [bins], ones)
                plsc.addupdate_scatter(sum2_v, [bins], v)
        return carry

    lax.fori_loop(0, VECS // hunroll, hist_body, 0)

    munroll = 8

    def merge_body(i, carry):
        for u in range(munroll):
            off = (i * munroll + u) * LANES
            sl = pl.ds(off, LANES)
            cnt_v[sl] = cnt_v[sl] + cnt2_v[sl]
            sum_v[sl] = sum_v[sl] + sum2_v[sl]
        return carry

    lax.fori_loop(0, BIN_VECS // munroll, merge_body, 0)

    pltpu.sync_copy(cnt_v, cnt_hbm.at[pl.ds(wid * NBINS, NBINS)])
    pltpu.sync_copy(sum_v, sum_hbm.at[pl.ds(wid * NBINS, NBINS)])


@functools.cache
def _sc_hist():
    return pl.kernel(
        _sc_hist_body,
        mesh=plsc.VectorSubcoreMesh(core_axis_name="c", subcore_axis_name="s"),
        out_type=[
            jax.ShapeDtypeStruct((NW * NBINS,), jnp.float32),
            jax.ShapeDtypeStruct((NW * NBINS,), jnp.float32),
        ],
        scratch_types=[
            pltpu.VMEM((CHUNK,), jnp.float32),
            pltpu.VMEM((NBINS,), jnp.float32),
            pltpu.VMEM((NBINS,), jnp.float32),
            pltpu.VMEM((NBINS,), jnp.float32),
            pltpu.VMEM((NBINS,), jnp.float32),
        ],
        compiler_params=pltpu.CompilerParams(needs_layout_passes=False),
    )


def _cumsum_shift(x, axis):
    n = x.shape[axis]
    sh = 1
    while sh < n:
        zeros = lax.slice_in_dim(jnp.zeros_like(x), 0, sh, axis=axis)
        shifted = lax.slice_in_dim(x, 0, n - sh, axis=axis)
        x = x + lax.concatenate([zeros, shifted], dimension=axis)
        sh *= 2
    return x


def _sel_body(c0_ref, s0_ref, o_ref):
    c = jnp.sum(c0_ref[...], axis=0)
    s = jnp.sum(s0_ref[...], axis=0)
    ce0 = _cumsum_shift(c, 0) - c
    row_off = jnp.sum(ce0, axis=1, keepdims=True)
    pe = (_cumsum_shift(c, 1) - c) + row_off
    total = jnp.sum(c)
    suf = total - pe
    r = lax.broadcasted_iota(jnp.int32, c.shape, 0)
    l = lax.broadcasted_iota(jnp.int32, c.shape, 1)
    bidx = r * 128 + l
    kf = jnp.float32(TOPK)
    bsel = jnp.max(jnp.where(suf >= kf, bidx, -1))
    above = bidx > bsel
    at = bidx == bsel
    c_above = jnp.sum(jnp.where(above, c, 0.0))
    s_above = jnp.sum(jnp.where(above, s, 0.0))
    c_bin = jnp.sum(jnp.where(at, c, 0.0))
    s_bin = jnp.sum(jnp.where(at, s, 0.0))
    m = kf - c_above
    res = (s_above + m * (s_bin / jnp.maximum(c_bin, 1.0))) / kf
    o_ref[...] = jnp.broadcast_to(res, (1, 1))


_sel = pl.pallas_call(
    _sel_body,
    in_specs=[
        pl.BlockSpec((NW, NBINS // 128, 128), lambda: (0, 0, 0))
        for _ in range(2)
    ],
    out_specs=pl.BlockSpec((1, 1), lambda: (0, 0)),
    out_shape=jax.ShapeDtypeStruct((1, 1), jnp.float32),
)


def kernel(preds, targets):
    t = targets.astype(jnp.int32)
    loss = _ce(preds, t).reshape(N_PIX)
    cnt, sm = _sc_hist()(loss)
    out = _sel(
        cnt.reshape(NW, NBINS // 128, 128), sm.reshape(NW, NBINS // 128, 128)
    )
    return out[0, 0]


# speedup vs baseline: 1.2456x; 1.0092x over previous
import functools

import jax
import jax.numpy as jnp
from jax import lax
from jax.experimental import pallas as pl
from jax.experimental.pallas import tpu as pltpu
from jax.experimental.pallas import tpu_sc as plsc

N_BATCH = 4
N_CLASSES = 19
SP = 512 * 512
N_PIX = N_BATCH * SP
TOPK = int(0.8 * N_PIX)

N_SPLIT = 1
B_SPLIT = N_BATCH // N_SPLIT
PIX_SPLIT = N_PIX // N_SPLIT

NW = 32
CHUNK = PIX_SPLIT // NW
LANES = 16
VECS = CHUNK // LANES
BIN_SHIFT = 17
NBINS = 1 << (31 - BIN_SHIFT)
BIN_VECS = NBINS // LANES


R_CE = 256


def _ce_body(p_ref, t_ref, o_ref):
    p = p_ref[...]
    t = t_ref[...][:, None, :, :]
    m = jnp.max(p, axis=1, keepdims=True)
    s = jnp.sum(jnp.exp(p - m), axis=1, keepdims=True)
    cls = lax.broadcasted_iota(jnp.int32, p.shape, 1)
    pt = jnp.sum(jnp.where(cls == t, p, 0.0), axis=1, keepdims=True)
    o_ref[...] = (m + jnp.log(s) - pt)[:, 0, :, :]


_ce = pl.pallas_call(
    _ce_body,
    grid=(B_SPLIT, 512 // R_CE),
    in_specs=[
        pl.BlockSpec((1, N_CLASSES, R_CE, 512), lambda b, j: (b, 0, j, 0)),
        pl.BlockSpec((1, R_CE, 512), lambda b, j: (b, j, 0)),
    ],
    out_specs=pl.BlockSpec((1, R_CE, 512), lambda b, j: (b, j, 0)),
    out_shape=jax.ShapeDtypeStruct((B_SPLIT, 512, 512), jnp.float32),
    compiler_params=pltpu.CompilerParams(
        dimension_semantics=("parallel", "parallel")
    ),
)


def _sc_hist_body(loss_hbm, cnt_hbm, sum_hbm, data_v, cnt_v, sum_v):
    wid = lax.axis_index("s") * 2 + lax.axis_index("c")
    pltpu.sync_copy(loss_hbm.at[pl.ds(wid * CHUNK, CHUNK)], data_v)

    zeros = jnp.zeros((LANES,), jnp.float32)
    zunroll = 16

    def zero_body(i, carry):
        for u in range(zunroll):
            off = (i * zunroll + u) * LANES
            cnt_v[pl.ds(off, LANES)] = zeros
            sum_v[pl.ds(off, LANES)] = zeros
        return carry

    lax.fori_loop(0, BIN_VECS // zunroll, zero_body, 0)

    ones = jnp.full((LANES,), 1.0, jnp.float32)
    izero = jnp.zeros((LANES,), jnp.int32)
    hunroll = 8

    def hist_body(i, carry):
        for u in range(hunroll):
            off = (i * hunroll + u) * LANES
            v = data_v[pl.ds(off, LANES)]
            bits = lax.bitcast_convert_type(v, jnp.int32)
            bins = lax.shift_right_logical(lax.max(bits, izero), BIN_SHIFT)
            plsc.addupdate_scatter(cnt_v, [bins], ones)
            plsc.addupdate_scatter(sum_v, [bins], v)
        return carry

    lax.fori_loop(0, VECS // hunroll, hist_body, 0)

    pltpu.sync_copy(cnt_v, cnt_hbm.at[pl.ds(wid * NBINS, NBINS)])
    pltpu.sync_copy(sum_v, sum_hbm.at[pl.ds(wid * NBINS, NBINS)])


@functools.cache
def _sc_hist():
    return pl.kernel(
        _sc_hist_body,
        mesh=plsc.VectorSubcoreMesh(core_axis_name="c", subcore_axis_name="s"),
        out_type=[
            jax.ShapeDtypeStruct((NW * NBINS,), jnp.float32),
            jax.ShapeDtypeStruct((NW * NBINS,), jnp.float32),
        ],
        scratch_types=[
            pltpu.VMEM((CHUNK,), jnp.float32),
            pltpu.VMEM((NBINS,), jnp.float32),
            pltpu.VMEM((NBINS,), jnp.float32),
        ],
        compiler_params=pltpu.CompilerParams(needs_layout_passes=False),
    )


def _cumsum_shift(x, axis):
    n = x.shape[axis]
    sh = 1
    while sh < n:
        zeros = lax.slice_in_dim(jnp.zeros_like(x), 0, sh, axis=axis)
        shifted = lax.slice_in_dim(x, 0, n - sh, axis=axis)
        x = x + lax.concatenate([zeros, shifted], dimension=axis)
        sh *= 2
    return x


def _sel_body(c0_ref, s0_ref, o_ref):
    c = jnp.sum(c0_ref[...], axis=0)
    s = jnp.sum(s0_ref[...], axis=0)
    ce0 = _cumsum_shift(c, 0) - c
    row_off = jnp.sum(ce0, axis=1, keepdims=True)
    pe = (_cumsum_shift(c, 1) - c) + row_off
    total = jnp.sum(c)
    suf = total - pe
    r = lax.broadcasted_iota(jnp.int32, c.shape, 0)
    l = lax.broadcasted_iota(jnp.int32, c.shape, 1)
    bidx = r * 128 + l
    kf = jnp.float32(TOPK)
    bsel = jnp.max(jnp.where(suf >= kf, bidx, -1))
    above = bidx > bsel
    at = bidx == bsel
    c_above = jnp.sum(jnp.where(above, c, 0.0))
    s_above = jnp.sum(jnp.where(above, s, 0.0))
    c_bin = jnp.sum(jnp.where(at, c, 0.0))
    s_bin = jnp.sum(jnp.where(at, s, 0.0))
    m = kf - c_above
    res = (s_above + m * (s_bin / jnp.maximum(c_bin, 1.0))) / kf
    o_ref[...] = jnp.broadcast_to(res, (1, 1))


_sel = pl.pallas_call(
    _sel_body,
    in_specs=[
        pl.BlockSpec((NW, NBINS // 128, 128), lambda: (0, 0, 0))
        for _ in range(2)
    ],
    out_specs=pl.BlockSpec((1, 1), lambda: (0, 0)),
    out_shape=jax.ShapeDtypeStruct((1, 1), jnp.float32),
)


def kernel(preds, targets):
    t = targets.astype(jnp.int32)
    loss = _ce(preds, t).reshape(N_PIX)
    cnt, sm = _sc_hist()(loss)
    out = _sel(
        cnt.reshape(NW, NBINS // 128, 128), sm.reshape(NW, NBINS // 128, 128)
    )
    return out[0, 0]


# speedup vs baseline: 1.4936x; 1.1991x over previous
import functools

import jax
import jax.numpy as jnp
from jax import lax
from jax.experimental import pallas as pl
from jax.experimental.pallas import tpu as pltpu
from jax.experimental.pallas import tpu_sc as plsc

N_BATCH = 4
N_CLASSES = 19
SP = 512 * 512
N_PIX = N_BATCH * SP
TOPK = int(0.8 * N_PIX)

N_SPLIT = 1
B_SPLIT = N_BATCH // N_SPLIT
PIX_SPLIT = N_PIX // N_SPLIT

NW = 32
CHUNK = PIX_SPLIT // NW
LANES = 16
VECS = CHUNK // LANES
BIN_SHIFT = 17
NBINS = 1 << (31 - BIN_SHIFT)
BIN_VECS = NBINS // LANES


R_CE = 256


def _ce_body(p_ref, t_ref, o_ref):
    p = p_ref[...]
    t = t_ref[...][:, None, :, :]
    m = jnp.max(p, axis=1, keepdims=True)
    s = jnp.sum(jnp.exp(p - m), axis=1, keepdims=True)
    cls = lax.broadcasted_iota(jnp.int32, p.shape, 1)
    pt = jnp.sum(jnp.where(cls == t, p, 0.0), axis=1, keepdims=True)
    o_ref[...] = (m + jnp.log(s) - pt)[:, 0, :, :]


_ce = pl.pallas_call(
    _ce_body,
    grid=(B_SPLIT, 512 // R_CE),
    in_specs=[
        pl.BlockSpec((1, N_CLASSES, R_CE, 512), lambda b, j: (b, 0, j, 0)),
        pl.BlockSpec((1, R_CE, 512), lambda b, j: (b, j, 0)),
    ],
    out_specs=pl.BlockSpec((1, R_CE, 512), lambda b, j: (b, j, 0)),
    out_shape=jax.ShapeDtypeStruct((B_SPLIT, 512, 512), jnp.float32),
    compiler_params=pltpu.CompilerParams(
        dimension_semantics=("parallel", "parallel")
    ),
)


def _sc_hist_body(loss_hbm, cnt_hbm, sum_hbm, data_v, cnt_v, sum_v):
    wid = lax.axis_index("s") * 2 + lax.axis_index("c")
    pltpu.sync_copy(loss_hbm.at[pl.ds(wid * CHUNK, CHUNK)], data_v)

    zeros = jnp.zeros((LANES,), jnp.float32)

    @plsc.parallel_loop(0, NBINS, LANES, unroll=8)
    def _(off):
        cnt_v[pl.ds(off, LANES)] = zeros
        sum_v[pl.ds(off, LANES)] = zeros

    ones = jnp.full((LANES,), 1.0, jnp.float32)
    izero = jnp.zeros((LANES,), jnp.int32)

    @plsc.parallel_loop(0, CHUNK, LANES, unroll=8)
    def _(off):
        v = data_v[pl.ds(off, LANES)]
        bits = lax.bitcast_convert_type(v, jnp.int32)
        bins = lax.shift_right_logical(lax.max(bits, izero), BIN_SHIFT)
        plsc.addupdate_scatter(cnt_v, [bins], ones)
        plsc.addupdate_scatter(sum_v, [bins], v)

    pltpu.sync_copy(cnt_v, cnt_hbm.at[pl.ds(wid * NBINS, NBINS)])
    pltpu.sync_copy(sum_v, sum_hbm.at[pl.ds(wid * NBINS, NBINS)])


@functools.cache
def _sc_hist():
    return pl.kernel(
        _sc_hist_body,
        mesh=plsc.VectorSubcoreMesh(core_axis_name="c", subcore_axis_name="s"),
        out_type=[
            jax.ShapeDtypeStruct((NW * NBINS,), jnp.float32),
            jax.ShapeDtypeStruct((NW * NBINS,), jnp.float32),
        ],
        scratch_types=[
            pltpu.VMEM((CHUNK,), jnp.float32),
            pltpu.VMEM((NBINS,), jnp.float32),
            pltpu.VMEM((NBINS,), jnp.float32),
        ],
        compiler_params=pltpu.CompilerParams(needs_layout_passes=False),
    )


def _cumsum_shift(x, axis):
    n = x.shape[axis]
    sh = 1
    while sh < n:
        zeros = lax.slice_in_dim(jnp.zeros_like(x), 0, sh, axis=axis)
        shifted = lax.slice_in_dim(x, 0, n - sh, axis=axis)
        x = x + lax.concatenate([zeros, shifted], dimension=axis)
        sh *= 2
    return x


def _sel_body(c0_ref, s0_ref, o_ref):
    c = jnp.sum(c0_ref[...], axis=0)
    s = jnp.sum(s0_ref[...], axis=0)
    ce0 = _cumsum_shift(c, 0) - c
    row_off = jnp.sum(ce0, axis=1, keepdims=True)
    pe = (_cumsum_shift(c, 1) - c) + row_off
    total = jnp.sum(c)
    suf = total - pe
    r = lax.broadcasted_iota(jnp.int32, c.shape, 0)
    l = lax.broadcasted_iota(jnp.int32, c.shape, 1)
    bidx = r * 128 + l
    kf = jnp.float32(TOPK)
    bsel = jnp.max(jnp.where(suf >= kf, bidx, -1))
    above = bidx > bsel
    at = bidx == bsel
    c_above = jnp.sum(jnp.where(above, c, 0.0))
    s_above = jnp.sum(jnp.where(above, s, 0.0))
    c_bin = jnp.sum(jnp.where(at, c, 0.0))
    s_bin = jnp.sum(jnp.where(at, s, 0.0))
    m = kf - c_above
    res = (s_above + m * (s_bin / jnp.maximum(c_bin, 1.0))) / kf
    o_ref[...] = jnp.broadcast_to(res, (1, 1))


_sel = pl.pallas_call(
    _sel_body,
    in_specs=[
        pl.BlockSpec((NW, NBINS // 128, 128), lambda: (0, 0, 0))
        for _ in range(2)
    ],
    out_specs=pl.BlockSpec((1, 1), lambda: (0, 0)),
    out_shape=jax.ShapeDtypeStruct((1, 1), jnp.float32),
)


def kernel(preds, targets):
    t = targets.astype(jnp.int32)
    loss = _ce(preds, t).reshape(N_PIX)
    cnt, sm = _sc_hist()(loss)
    out = _sel(
        cnt.reshape(NW, NBINS // 128, 128), sm.reshape(NW, NBINS // 128, 128)
    )
    return out[0, 0]


# speedup vs baseline: 1.5185x; 1.0167x over previous
import functools

import jax
import jax.numpy as jnp
from jax import lax
from jax.experimental import pallas as pl
from jax.experimental.pallas import tpu as pltpu
from jax.experimental.pallas import tpu_sc as plsc

N_BATCH = 4
N_CLASSES = 19
SP = 512 * 512
N_PIX = N_BATCH * SP
TOPK = int(0.8 * N_PIX)

N_SPLIT = 1
B_SPLIT = N_BATCH // N_SPLIT
PIX_SPLIT = N_PIX // N_SPLIT

NW = 32
CHUNK = PIX_SPLIT // NW
LANES = 16
VECS = CHUNK // LANES
BIN_SHIFT = 17
NBINS = 1 << (31 - BIN_SHIFT)
BIN_VECS = NBINS // LANES


R_CE = 256


def _ce_body(p_ref, t_ref, o_ref):
    p = p_ref[...]
    t = t_ref[...][:, None, :, :]
    m = jnp.max(p, axis=1, keepdims=True)
    s = jnp.sum(jnp.exp(p - m), axis=1, keepdims=True)
    cls = lax.broadcasted_iota(jnp.int32, p.shape, 1)
    pt = jnp.sum(jnp.where(cls == t, p, 0.0), axis=1, keepdims=True)
    o_ref[...] = (m + jnp.log(s) - pt)[:, 0, :, :]


_ce = pl.pallas_call(
    _ce_body,
    grid=(B_SPLIT, 512 // R_CE),
    in_specs=[
        pl.BlockSpec((1, N_CLASSES, R_CE, 512), lambda b, j: (b, 0, j, 0)),
        pl.BlockSpec((1, R_CE, 512), lambda b, j: (b, j, 0)),
    ],
    out_specs=pl.BlockSpec((1, R_CE, 512), lambda b, j: (b, j, 0)),
    out_shape=jax.ShapeDtypeStruct((B_SPLIT, 512, 512), jnp.float32),
    compiler_params=pltpu.CompilerParams(
        dimension_semantics=("parallel", "parallel")
    ),
)


def _sc_hist_body(loss_hbm, cnt_hbm, sum_hbm, sem, data_v, cnt_v, sum_v):
    wid = lax.axis_index("s") * 2 + lax.axis_index("c")
    copy = pltpu.async_copy(
        loss_hbm.at[pl.ds(wid * CHUNK, CHUNK)], data_v, sem
    )

    zeros = jnp.zeros((LANES,), jnp.float32)

    @plsc.parallel_loop(0, NBINS, LANES, unroll=8)
    def _(off):
        cnt_v[pl.ds(off, LANES)] = zeros
        sum_v[pl.ds(off, LANES)] = zeros

    copy.wait()

    ones = jnp.full((LANES,), 1.0, jnp.float32)
    izero = jnp.zeros((LANES,), jnp.int32)

    @plsc.parallel_loop(0, CHUNK, LANES, unroll=16)
    def _(off):
        v = data_v[pl.ds(off, LANES)]
        bits = lax.bitcast_convert_type(v, jnp.int32)
        bins = lax.shift_right_logical(lax.max(bits, izero), BIN_SHIFT)
        plsc.addupdate_scatter(cnt_v, [bins], ones)
        plsc.addupdate_scatter(sum_v, [bins], v)

    pltpu.sync_copy(cnt_v, cnt_hbm.at[pl.ds(wid * NBINS, NBINS)])
    pltpu.sync_copy(sum_v, sum_hbm.at[pl.ds(wid * NBINS, NBINS)])


@functools.cache
def _sc_hist():
    return pl.kernel(
        _sc_hist_body,
        mesh=plsc.VectorSubcoreMesh(core_axis_name="c", subcore_axis_name="s"),
        out_type=[
            jax.ShapeDtypeStruct((NW * NBINS,), jnp.float32),
            jax.ShapeDtypeStruct((NW * NBINS,), jnp.float32),
        ],
        scratch_types=[
            pltpu.SemaphoreType.DMA,
            pltpu.VMEM((CHUNK,), jnp.float32),
            pltpu.VMEM((NBINS,), jnp.float32),
            pltpu.VMEM((NBINS,), jnp.float32),
        ],
        compiler_params=pltpu.CompilerParams(needs_layout_passes=False),
    )


def _cumsum_shift(x, axis):
    n = x.shape[axis]
    sh = 1
    while sh < n:
        zeros = lax.slice_in_dim(jnp.zeros_like(x), 0, sh, axis=axis)
        shifted = lax.slice_in_dim(x, 0, n - sh, axis=axis)
        x = x + lax.concatenate([zeros, shifted], dimension=axis)
        sh *= 2
    return x


def _sel_body(c0_ref, s0_ref, o_ref):
    c = jnp.sum(c0_ref[...], axis=0)
    s = jnp.sum(s0_ref[...], axis=0)
    ce0 = _cumsum_shift(c, 0) - c
    row_off = jnp.sum(ce0, axis=1, keepdims=True)
    pe = (_cumsum_shift(c, 1) - c) + row_off
    total = jnp.sum(c)
    suf = total - pe
    r = lax.broadcasted_iota(jnp.int32, c.shape, 0)
    l = lax.broadcasted_iota(jnp.int32, c.shape, 1)
    bidx = r * 128 + l
    kf = jnp.float32(TOPK)
    bsel = jnp.max(jnp.where(suf >= kf, bidx, -1))
    above = bidx > bsel
    at = bidx == bsel
    c_above = jnp.sum(jnp.where(above, c, 0.0))
    s_above = jnp.sum(jnp.where(above, s, 0.0))
    c_bin = jnp.sum(jnp.where(at, c, 0.0))
    s_bin = jnp.sum(jnp.where(at, s, 0.0))
    m = kf - c_above
    res = (s_above + m * (s_bin / jnp.maximum(c_bin, 1.0))) / kf
    o_ref[...] = jnp.broadcast_to(res, (1, 1))


_sel = pl.pallas_call(
    _sel_body,
    in_specs=[
        pl.BlockSpec((NW, NBINS // 128, 128), lambda: (0, 0, 0))
        for _ in range(2)
    ],
    out_specs=pl.BlockSpec((1, 1), lambda: (0, 0)),
    out_shape=jax.ShapeDtypeStruct((1, 1), jnp.float32),
)


def kernel(preds, targets):
    t = targets.astype(jnp.int32)
    loss = _ce(preds, t).reshape(N_PIX)
    cnt, sm = _sc_hist()(loss)
    out = _sel(
        cnt.reshape(NW, NBINS // 128, 128), sm.reshape(NW, NBINS // 128, 128)
    )
    return out[0, 0]
